# baseline (speedup 1.0000x reference)
import functools
import jax
import jax.numpy as jnp
from jax import lax
from jax.experimental import pallas as pl
from jax.experimental.pallas import tpu as pltpu, tpu_sc as plsc

B = 16384
K = 64
NW = 32
RW = B // NW
G = RW // 16


def _sc_dot(gut, git):
    mesh = plsc.VectorSubcoreMesh(core_axis_name="c", subcore_axis_name="s")

    @functools.partial(
        pl.kernel,
        mesh=mesh,
        out_type=jax.ShapeDtypeStruct((B,), jnp.float32),
        scratch_types=[
            pltpu.VMEM((K, RW), jnp.float32),
            pltpu.VMEM((K, RW), jnp.float32),
            pltpu.VMEM((RW,), jnp.float32),
        ],
    )
    def k(gut_hbm, git_hbm, xui_hbm, ubuf, vbuf, xbuf):
        wid = lax.axis_index("s") * 2 + lax.axis_index("c")
        base = wid * RW
        pltpu.sync_copy(gut_hbm.at[:, pl.ds(base, RW)], ubuf)
        pltpu.sync_copy(git_hbm.at[:, pl.ds(base, RW)], vbuf)

        def group(j, _):
            col = j * 16
            acc = jnp.zeros((16,), jnp.float32)
            for kk in range(K):
                acc = acc + ubuf[kk, pl.ds(col, 16)] * vbuf[kk, pl.ds(col, 16)]
            xbuf[pl.ds(col, 16)] = acc
            return 0

        lax.fori_loop(0, G, group, 0)
        pltpu.sync_copy(xbuf, xui_hbm.at[pl.ds(base, RW)])

    return k(gut, git)


def kernel(gu, gi):
    xui = _sc_dot(gu.T, gi.T)
    return (xui, gu, gi)


# speedup vs baseline: 1.0566x; 1.0566x over previous
import functools
import jax
import jax.numpy as jnp
from jax import lax
from jax.experimental import pallas as pl
from jax.experimental.pallas import tpu as pltpu, tpu_sc as plsc

B = 16384
K = 64
HS = 4096
NW = 32
RW = HS // NW
G = RW // 16
CB = 2048
NTC = (B - HS) // CB


def _sc_dot(gut, git):
    mesh = plsc.VectorSubcoreMesh(core_axis_name="c", subcore_axis_name="s")

    @functools.partial(
        pl.kernel,
        mesh=mesh,
        out_type=jax.ShapeDtypeStruct((HS,), jnp.float32),
        scratch_types=[
            pltpu.VMEM((K, RW), jnp.float32),
            pltpu.VMEM((K, RW), jnp.float32),
            pltpu.VMEM((RW,), jnp.float32),
        ],
    )
    def k(gut_hbm, git_hbm, xui_hbm, ubuf, vbuf, xbuf):
        wid = lax.axis_index("s") * 2 + lax.axis_index("c")
        base = wid * RW
        pltpu.sync_copy(gut_hbm.at[:, pl.ds(base, RW)], ubuf)
        pltpu.sync_copy(git_hbm.at[:, pl.ds(base, RW)], vbuf)

        def group(j, _):
            col = j * 16
            acc = jnp.zeros((16,), jnp.float32)
            for kk in range(K):
                acc = acc + ubuf[kk, pl.ds(col, 16)] * vbuf[kk, pl.ds(col, 16)]
            xbuf[pl.ds(col, 16)] = acc
            return 0

        lax.fori_loop(0, G, group, 0)
        pltpu.sync_copy(xbuf, xui_hbm.at[pl.ds(base, RW)])

    return k(gut, git)


def _tc_body(gu_ref, gi_ref, xui_ref):
    xui_ref[...] = jnp.sum(gu_ref[...] * gi_ref[...], axis=0)


def _tc_dot(gut, git):
    off = HS // CB
    return pl.pallas_call(
        _tc_body,
        grid=(NTC,),
        in_specs=[
            pl.BlockSpec((K, CB), lambda i: (0, i + off)),
            pl.BlockSpec((K, CB), lambda i: (0, i + off)),
        ],
        out_specs=pl.BlockSpec((CB,), lambda i: (i,)),
        out_shape=jax.ShapeDtypeStruct((B - HS,), jnp.float32),
    )(gut, git)


def kernel(gu, gi):
    gut = gu.T
    git = gi.T
    xui_lo = _sc_dot(gut, git)
    xui_hi = _tc_dot(gut, git)
    return (jnp.concatenate([xui_lo, xui_hi]), gu, gi)


# speedup vs baseline: 2.4939x; 2.3603x over previous
import jax
import jax.numpy as jnp
from jax.experimental import pallas as pl
from jax.experimental.pallas import tpu as pltpu

B = 16384
K = 64
CH = 2048
N = B // CH
D = 4
P = 2


def _body(gu_hbm, gi_hbm, xui_hbm, ubuf, vbuf, xbuf, uin, vin, xsem):
    def start_in(c):
        s = c % D
        pltpu.make_async_copy(gu_hbm.at[:, pl.ds(c * CH, CH)], ubuf.at[s],
                              uin.at[s]).start(priority=0)
        pltpu.make_async_copy(gi_hbm.at[:, pl.ds(c * CH, CH)], vbuf.at[s],
                              vin.at[s]).start(priority=1)

    def wait_in(c):
        s = c % D
        pltpu.make_async_copy(gu_hbm.at[:, pl.ds(c * CH, CH)], ubuf.at[s],
                              uin.at[s]).wait()
        pltpu.make_async_copy(gi_hbm.at[:, pl.ds(c * CH, CH)], vbuf.at[s],
                              vin.at[s]).wait()

    for c in range(P):
        start_in(c)

    for c in range(N):
        s = c % D
        wait_in(c)
        if c + P < N:
            start_in(c + P)
        xbuf[pl.ds(c * CH, CH)] = jnp.sum(ubuf[s] * vbuf[s], axis=0)

    cp = pltpu.make_async_copy(xbuf, xui_hbm, xsem)
    cp.start()
    cp.wait()


def kernel(gu, gi):
    xui = pl.pallas_call(
        _body,
        in_specs=[
            pl.BlockSpec(memory_space=pl.ANY),
            pl.BlockSpec(memory_space=pl.ANY),
        ],
        out_specs=pl.BlockSpec(memory_space=pl.ANY),
        out_shape=jax.ShapeDtypeStruct((B,), gu.dtype),
        scratch_shapes=[
            pltpu.VMEM((D, K, CH), jnp.float32),
            pltpu.VMEM((D, K, CH), jnp.float32),
            pltpu.VMEM((B,), jnp.float32),
            pltpu.SemaphoreType.DMA((D,)),
            pltpu.SemaphoreType.DMA((D,)),
            pltpu.SemaphoreType.DMA,
        ],
    )(gu.T, gi.T)
    return (xui, gu, gi)
